# baseline (speedup 1.0000x reference)
import functools

import jax
import jax.numpy as jnp
from jax import lax
from jax.experimental import pallas as pl
from jax.experimental.pallas import tpu as pltpu
from jax.experimental.pallas import tpu_sc as plsc

_N = 50000
_E = 800000
_DH = 128
_G = 512
_NC = 2
_NS = 16
_NW = _NC * _NS
_EPT = _E // _NW
_CH = 8448
_NCK = 6
_NPAD = _NCK * _CH
_CAP = 26624
_K = 128
_SHARE = _CH // _NS

_mesh = plsc.VectorSubcoreMesh(core_axis_name="c", subcore_axis_name="s")


def _row_copy_loop(src_ref, dst_ref, n, src_base, dst_base, src_advances=True):
    off = 0
    while off < n:
        blk = min(128, n - off)
        soff = src_base + off if src_advances else src_base
        pltpu.sync_copy(src_ref.at[pl.ds(soff, blk)],
                        dst_ref.at[pl.ds(dst_base + off, blk)])
        off += blk


@functools.partial(
    pl.kernel,
    out_type=(
        jax.ShapeDtypeStruct((_NW * _NCK * _CAP,), jnp.int32),
        jax.ShapeDtypeStruct((_NW * _NCK * _CAP,), jnp.int32),
        jax.ShapeDtypeStruct((_NW, 16), jnp.int32),
    ),
    mesh=_mesh,
    compiler_params=pltpu.CompilerParams(needs_layout_passes=False),
    scratch_types=[
        pltpu.VMEM((_EPT + 16,), jnp.int32),
        pltpu.VMEM((_EPT + 16,), jnp.int32),
        pltpu.VMEM((_CAP + 128,), jnp.int32),
        pltpu.VMEM((_CAP + 128,), jnp.int32),
        pltpu.VMEM((16,), jnp.int32),
    ],
)
def _bucket(src_hbm, dst_hbm, sl_hbm, dl_hbm, cnt_hbm,
            srcv, dstv, sbuf, dbuf, cntv):
    c = lax.axis_index("c")
    s = lax.axis_index("s")
    wid = s * _NC + c
    base_e = wid * _EPT
    pltpu.sync_copy(src_hbm.at[pl.ds(base_e, _EPT)], srcv.at[pl.ds(0, _EPT)])
    pltpu.sync_copy(dst_hbm.at[pl.ds(base_e, _EPT)], dstv.at[pl.ds(0, _EPT)])
    iota = lax.iota(jnp.int32, 16)
    cnts = jnp.zeros((16,), jnp.int32)
    nfull = _EPT // 16
    ntail = _EPT - nfull * 16

    for b in range(_NCK):
        def scan_step(cur, d, sv, extra_mask):
            m = (d >= b * _CH) & (d < (b + 1) * _CH) & extra_mask
            plsc.store_compressed(sbuf.at[pl.ds(cur, 16)], sv, mask=m)
            plsc.store_compressed(dbuf.at[pl.ds(cur, 16)], d - b * _CH, mask=m)
            return cur + jnp.max(plsc.all_reduce_population_count(m))

        def scan_body(i, cur):
            d = dstv[pl.ds(i * 16, 16)]
            sv = srcv[pl.ds(i * 16, 16)]
            return scan_step(cur, d, sv, jnp.full((16,), True))

        cnt = lax.fori_loop(0, nfull, scan_body, jnp.int32(0))
        d = dstv[pl.ds(nfull * 16, 16)]
        sv = srcv[pl.ds(nfull * 16, 16)]
        cnt = scan_step(cnt, d, sv, iota < ntail)
        npad = ((cnt + _K - 1) // _K) * _K
        dummy_s = jnp.zeros((16,), jnp.int32)
        dummy_d = jnp.full((16,), _CH, jnp.int32)

        def pad_body(j, _):
            offp = cnt + j * 16
            k = jnp.clip(npad - offp, 0, 16)
            m = iota < k
            plsc.store_compressed(sbuf.at[pl.ds(offp, 16)], dummy_s, mask=m)
            plsc.store_compressed(dbuf.at[pl.ds(offp, 16)], dummy_d, mask=m)
            return 0

        lax.fori_loop(0, _K // 16, pad_body, 0)
        lbase = (wid * _NCK + b) * _CAP

        def dma_body(i, _):
            pltpu.sync_copy(sbuf.at[pl.ds(i * _K, _K)],
                            sl_hbm.at[pl.ds(lbase + i * _K, _K)])
            pltpu.sync_copy(dbuf.at[pl.ds(i * _K, _K)],
                            dl_hbm.at[pl.ds(lbase + i * _K, _K)])
            return 0

        lax.fori_loop(0, npad // _K, dma_body, 0)
        cnts = jnp.where(iota == b, npad, cnts)

    cntv[...] = cnts
    pltpu.sync_copy(cntv, cnt_hbm.at[wid])


_CAPB = _CAP // _K
_W = 16


@functools.lru_cache(None)
def _make_agg(d):
    @functools.partial(
        pl.kernel,
        out_type=jax.ShapeDtypeStruct((_NPAD, d), jnp.float32),
        mesh=_mesh,
        compiler_params=pltpu.CompilerParams(needs_layout_passes=False),
        scratch_types=[
            pltpu.VMEM((_NW, 16), jnp.int32),
            pltpu.VMEM((_W, _K), jnp.int32),
            pltpu.VMEM((_W, _K), jnp.int32),
            pltpu.VMEM((_K, d), jnp.float32),
            pltpu.VMEM((_K, d), jnp.float32),
            pltpu.VMEM_SHARED((_CH + 1, d), jnp.float32),
            pltpu.SemaphoreType.DMA,
            pltpu.SemaphoreType.DMA,
        ],
    )
    def agg(h_hbm, sl_hbm, dl_hbm, cnt_hbm, z_hbm, agg_hbm,
            cntv, sidx, didx, rows0, rows1, acc, sem0, sem1):
        c = lax.axis_index("c")
        s = lax.axis_index("s")
        iota = lax.iota(jnp.int32, 16)
        pltpu.sync_copy(cnt_hbm, cntv)
        for cc in range(_NCK // _NC):
            chunk = c * (_NCK // _NC) + cc
            base_row = chunk * _CH
            r0 = s * _SHARE
            _row_copy_loop(z_hbm, acc, _SHARE, 0, r0, src_advances=False)
            plsc.subcore_barrier()
            for li in range(_NW // _NS):
                t_src = s * (_NW // _NS) + li
                cv = cntv[t_src]
                npad = jnp.max(jnp.where(iota == chunk, cv, 0))
                nblk = npad // _K
                lrow = (t_src * _NCK + chunk) * _CAPB

                def win_body(w, _):
                    wb0 = w * _W
                    jb = jnp.minimum(_W, nblk - wb0)
                    pltpu.sync_copy(sl_hbm.at[pl.ds(lrow + wb0, _W)], sidx)
                    pltpu.sync_copy(dl_hbm.at[pl.ds(lrow + wb0, _W)], didx)
                    pltpu.async_copy(h_hbm.at[sidx.at[0]], rows0, sem0)

                    def pair_body(i, _):
                        j0 = 2 * i
                        j1 = 2 * i + 1

                        @pl.when(j1 < jb)
                        def _():
                            pltpu.async_copy(h_hbm.at[sidx.at[j1]], rows1, sem1)

                        pltpu.make_async_copy(h_hbm.at[sidx.at[j0]], rows0,
                                              sem0).wait()
                        pltpu.sync_copy(rows0, acc.at[didx.at[j0]], add=True)

                        @pl.when(j1 < jb)
                        def _():
                            @pl.when(j1 + 1 < jb)
                            def _():
                                pltpu.async_copy(h_hbm.at[sidx.at[j1 + 1]],
                                                 rows0, sem0)
                            pltpu.make_async_copy(h_hbm.at[sidx.at[j1]], rows1,
                                                  sem1).wait()
                            pltpu.sync_copy(rows1, acc.at[didx.at[j1]], add=True)
                        return 0

                    lax.fori_loop(0, (jb + 1) // 2, pair_body, 0)
                    return 0

                lax.fori_loop(0, (nblk + _W - 1) // _W, win_body, 0)
            plsc.subcore_barrier()
            _row_copy_loop(acc, agg_hbm, _SHARE, r0, base_row + r0)
            plsc.subcore_barrier()

    return agg


_NFULL = _N // _K
_NTAIL = _N - _NFULL * _K


@functools.partial(
    pl.kernel,
    out_type=jax.ShapeDtypeStruct((_NC * _G, _DH), jnp.float32),
    mesh=_mesh,
    compiler_params=pltpu.CompilerParams(needs_layout_passes=False),
    scratch_types=[
        pltpu.VMEM((_K,), jnp.int32),
        pltpu.VMEM((_NTAIL,), jnp.int32),
        pltpu.VMEM((_K, _DH), jnp.float32),
        pltpu.VMEM((_NTAIL, _DH), jnp.float32),
        pltpu.VMEM_SHARED((_G, _DH), jnp.float32),
        pltpu.SemaphoreType.DMA,
    ],
)
def _pool(h_hbm, b_hbm, z_hbm, p_hbm, bidx, bidxt, rows, rowst, acc, sem):
    c = lax.axis_index("c")
    s = lax.axis_index("s")
    wid = s * _NC + c
    gshare = _G // _NS
    pltpu.sync_copy(z_hbm.at[pl.ds(0, gshare)], acc.at[pl.ds(s * gshare, gshare)])
    plsc.subcore_barrier()
    nblk = (_NFULL - wid + _NW - 1) // _NW

    def body(i, _):
        r0 = (i * _NW + wid) * _K
        pltpu.sync_copy(b_hbm.at[pl.ds(r0, _K)], bidx)
        pltpu.sync_copy(h_hbm.at[pl.ds(r0, _K)], rows)
        pltpu.sync_copy(rows, acc.at[bidx], add=True)
        return 0

    lax.fori_loop(0, nblk, body, 0)

    @pl.when(wid == 0)
    def _():
        pltpu.sync_copy(b_hbm.at[pl.ds(_NFULL * _K, _NTAIL)], bidxt)
        pltpu.sync_copy(h_hbm.at[pl.ds(_NFULL * _K, _NTAIL)], rowst)
        pltpu.sync_copy(rowst, acc.at[bidxt], add=True)

    plsc.subcore_barrier()
    pltpu.sync_copy(acc.at[pl.ds(s * gshare, gshare)],
                    p_hbm.at[pl.ds(c * _G + s * gshare, gshare)])


_RB = 2000
_NBLK = _N // _RB


@functools.lru_cache(None)
def _make_layer_a(din):
    def body(h_ref, a_ref, w_ref, b_ref, y_ref, s_ref, q_ref):
        t = h_ref[...] + a_ref[...]
        y = jnp.dot(t, w_ref[...], preferred_element_type=jnp.float32) + b_ref[...]
        y_ref[...] = y

        @pl.when(pl.program_id(0) == 0)
        def _():
            s_ref[...] = jnp.zeros_like(s_ref)
            q_ref[...] = jnp.zeros_like(q_ref)

        s_ref[...] += jnp.sum(y, axis=0, keepdims=True)
        q_ref[...] += jnp.sum(y * y, axis=0, keepdims=True)

    return pl.pallas_call(
        body,
        grid=(_NBLK,),
        in_specs=[
            pl.BlockSpec((_RB, din), lambda i: (i, 0)),
            pl.BlockSpec((_RB, din), lambda i: (i, 0)),
            pl.BlockSpec((din, _DH), lambda i: (0, 0)),
            pl.BlockSpec((1, _DH), lambda i: (0, 0)),
        ],
        out_specs=[
            pl.BlockSpec((_RB, _DH), lambda i: (i, 0)),
            pl.BlockSpec((1, _DH), lambda i: (0, 0)),
            pl.BlockSpec((1, _DH), lambda i: (0, 0)),
        ],
        out_shape=[
            jax.ShapeDtypeStruct((_N, _DH), jnp.float32),
            jax.ShapeDtypeStruct((1, _DH), jnp.float32),
            jax.ShapeDtypeStruct((1, _DH), jnp.float32),
        ],
    )


def _layer_b_body(y_ref, s_ref, q_ref, g_ref, be_ref, w_ref, bb_ref, o_ref):
    m = s_ref[...] / _N
    v = q_ref[...] / _N - m * m
    yn = (y_ref[...] - m) / jnp.sqrt(v + 1e-5) * g_ref[...] + be_ref[...]
    r = jnp.maximum(yn, 0.0)
    h2 = jnp.dot(r, w_ref[...], preferred_element_type=jnp.float32) + bb_ref[...]
    o_ref[...] = jnp.maximum(h2, 0.0)


_layer_b = pl.pallas_call(
    _layer_b_body,
    grid=(_NBLK,),
    in_specs=[
        pl.BlockSpec((_RB, _DH), lambda i: (i, 0)),
        pl.BlockSpec((1, _DH), lambda i: (0, 0)),
        pl.BlockSpec((1, _DH), lambda i: (0, 0)),
        pl.BlockSpec((1, _DH), lambda i: (0, 0)),
        pl.BlockSpec((1, _DH), lambda i: (0, 0)),
        pl.BlockSpec((_DH, _DH), lambda i: (0, 0)),
        pl.BlockSpec((1, _DH), lambda i: (0, 0)),
    ],
    out_specs=pl.BlockSpec((_RB, _DH), lambda i: (i, 0)),
    out_shape=jax.ShapeDtypeStruct((_N, _DH), jnp.float32),
)


def _head_body(p_ref, w1_ref, b1_ref, w2_ref, b2_ref, o_ref):
    hg = p_ref[:_G, :] + p_ref[_G:, :]
    z = jnp.maximum(
        jnp.dot(hg, w1_ref[...], preferred_element_type=jnp.float32) + b1_ref[...],
        0.0)
    o_ref[...] = jnp.dot(z, w2_ref[...], preferred_element_type=jnp.float32) + b2_ref[...]


_head = pl.pallas_call(
    _head_body,
    out_shape=jax.ShapeDtypeStruct((_G, _DH), jnp.float32),
)


def kernel(x, edge_index, batch,
           w1a, b1a, g1, be1, w1b, b1b,
           w2a, b2a, g2, be2, w2b, b2b,
           w3a, b3a, g3, be3, w3b, b3b,
           wl1, bl1, wl2, bl2):
    src = edge_index[0]
    dst = edge_index[1]
    x_pad = jnp.pad(x, ((0, 0), (0, _DH - 11)))
    w1a_p = jnp.pad(w1a, ((0, _DH - 11), (0, 0)))
    z128 = jnp.zeros((_K, _DH), jnp.float32)

    sl, dl, cnts = _bucket(src, dst)

    sl2 = sl.reshape(-1, _K)
    dl2 = dl.reshape(-1, _K)

    def gin_layer(h, din, wa, ba, g, be, wb, bb, z):
        aggv = _make_agg(din)(h, sl2, dl2, cnts, z)
        y, sv, qv = _make_layer_a(din)(h, aggv, wa, ba.reshape(1, -1))
        return _layer_b(y, sv, qv, g.reshape(1, -1), be.reshape(1, -1),
                        wb, bb.reshape(1, -1))

    h1 = gin_layer(x_pad, _DH, w1a_p, b1a, g1, be1, w1b, b1b, z128)
    h2 = gin_layer(h1, _DH, w2a, b2a, g2, be2, w2b, b2b, z128)
    h3 = gin_layer(h2, _DH, w3a, b3a, g3, be3, w3b, b3b, z128)

    p = _pool(h3, batch, z128)
    wl2_p = jnp.pad(wl2, ((0, 0), (0, _DH - 1)))
    bl2_p = jnp.pad(bl2, ((0, _DH - 1))).reshape(1, -1)
    o = _head(p, wl1, bl1.reshape(1, -1), wl2_p, bl2_p)
    return o[:, 0:1]


# speedup vs baseline: 1.0473x; 1.0473x over previous
import functools

import jax
import jax.numpy as jnp
from jax import lax
from jax.experimental import pallas as pl
from jax.experimental.pallas import tpu as pltpu
from jax.experimental.pallas import tpu_sc as plsc

_N = 50000
_E = 800000
_DH = 128
_G = 512
_NC = 2
_NS = 16
_NW = _NC * _NS
_EPT = _E // _NW
_CH = 12544
_NCK = 4
_NPAD = _NCK * _CH
_CAP = 26624
_K = 128
_SHARE = _CH // _NS

_mesh = plsc.VectorSubcoreMesh(core_axis_name="c", subcore_axis_name="s")


def _row_copy_loop(src_ref, dst_ref, n, src_base, dst_base, src_advances=True):
    off = 0
    while off < n:
        blk = min(128, n - off)
        soff = src_base + off if src_advances else src_base
        pltpu.sync_copy(src_ref.at[pl.ds(soff, blk)],
                        dst_ref.at[pl.ds(dst_base + off, blk)])
        off += blk


@functools.partial(
    pl.kernel,
    out_type=(
        jax.ShapeDtypeStruct((_NW * _NCK * _CAP,), jnp.int32),
        jax.ShapeDtypeStruct((_NW * _NCK * _CAP,), jnp.int32),
        jax.ShapeDtypeStruct((_NW, 16), jnp.int32),
    ),
    mesh=_mesh,
    compiler_params=pltpu.CompilerParams(needs_layout_passes=False),
    scratch_types=[
        pltpu.VMEM((_EPT + 16,), jnp.int32),
        pltpu.VMEM((_EPT + 16,), jnp.int32),
        pltpu.VMEM((_CAP + 128,), jnp.int32),
        pltpu.VMEM((_CAP + 128,), jnp.int32),
        pltpu.VMEM((16,), jnp.int32),
    ],
)
def _bucket(src_hbm, dst_hbm, sl_hbm, dl_hbm, cnt_hbm,
            srcv, dstv, sbuf, dbuf, cntv):
    c = lax.axis_index("c")
    s = lax.axis_index("s")
    wid = s * _NC + c
    base_e = wid * _EPT
    pltpu.sync_copy(src_hbm.at[pl.ds(base_e, _EPT)], srcv.at[pl.ds(0, _EPT)])
    pltpu.sync_copy(dst_hbm.at[pl.ds(base_e, _EPT)], dstv.at[pl.ds(0, _EPT)])
    iota = lax.iota(jnp.int32, 16)
    cnts = jnp.zeros((16,), jnp.int32)
    nfull = _EPT // 16
    ntail = _EPT - nfull * 16

    for b in range(_NCK):
        def scan_step(cur, d, sv, extra_mask):
            m = (d >= b * _CH) & (d < (b + 1) * _CH) & extra_mask
            plsc.store_compressed(sbuf.at[pl.ds(cur, 16)], sv, mask=m)
            plsc.store_compressed(dbuf.at[pl.ds(cur, 16)], d - b * _CH, mask=m)
            return cur + jnp.max(plsc.all_reduce_population_count(m))

        def scan_body(i, cur):
            d = dstv[pl.ds(i * 16, 16)]
            sv = srcv[pl.ds(i * 16, 16)]
            return scan_step(cur, d, sv, jnp.full((16,), True))

        cnt = lax.fori_loop(0, nfull, scan_body, jnp.int32(0))
        d = dstv[pl.ds(nfull * 16, 16)]
        sv = srcv[pl.ds(nfull * 16, 16)]
        cnt = scan_step(cnt, d, sv, iota < ntail)
        npad = ((cnt + _K - 1) // _K) * _K
        dummy_s = jnp.zeros((16,), jnp.int32)
        dummy_d = jnp.full((16,), _CH, jnp.int32)

        def pad_body(j, _):
            offp = cnt + j * 16
            k = jnp.clip(npad - offp, 0, 16)
            m = iota < k
            plsc.store_compressed(sbuf.at[pl.ds(offp, 16)], dummy_s, mask=m)
            plsc.store_compressed(dbuf.at[pl.ds(offp, 16)], dummy_d, mask=m)
            return 0

        lax.fori_loop(0, _K // 16, pad_body, 0)
        lbase = (wid * _NCK + b) * _CAP

        nbig = npad // 1024

        def dma_big(i, _):
            pltpu.sync_copy(sbuf.at[pl.ds(i * 1024, 1024)],
                            sl_hbm.at[pl.ds(lbase + i * 1024, 1024)])
            pltpu.sync_copy(dbuf.at[pl.ds(i * 1024, 1024)],
                            dl_hbm.at[pl.ds(lbase + i * 1024, 1024)])
            return 0

        lax.fori_loop(0, nbig, dma_big, 0)

        def dma_tail(i, _):
            off2 = nbig * 1024 + i * _K
            pltpu.sync_copy(sbuf.at[pl.ds(off2, _K)],
                            sl_hbm.at[pl.ds(lbase + off2, _K)])
            pltpu.sync_copy(dbuf.at[pl.ds(off2, _K)],
                            dl_hbm.at[pl.ds(lbase + off2, _K)])
            return 0

        lax.fori_loop(0, (npad - nbig * 1024) // _K, dma_tail, 0)
        cnts = jnp.where(iota == b, npad, cnts)

    cntv[...] = cnts
    pltpu.sync_copy(cntv, cnt_hbm.at[wid])


_CAPB = _CAP // _K
_W = 16


@functools.lru_cache(None)
def _make_agg(d):
    @functools.partial(
        pl.kernel,
        out_type=jax.ShapeDtypeStruct((_NPAD, d), jnp.float32),
        mesh=_mesh,
        compiler_params=pltpu.CompilerParams(needs_layout_passes=False),
        scratch_types=[
            pltpu.VMEM((_NW, 16), jnp.int32),
            pltpu.VMEM((_W, _K), jnp.int32),
            pltpu.VMEM((_W, _K), jnp.int32),
            pltpu.VMEM((_K, d), jnp.float32),
            pltpu.VMEM_SHARED((_CH + 1, d), jnp.float32),
            pltpu.SemaphoreType.DMA,
        ],
    )
    def agg(h_hbm, sl_hbm, dl_hbm, cnt_hbm, z_hbm, agg_hbm,
            cntv, sidx, didx, rows0, acc, sem0):
        c = lax.axis_index("c")
        s = lax.axis_index("s")
        iota = lax.iota(jnp.int32, 16)
        pltpu.sync_copy(cnt_hbm, cntv)
        for cc in range(_NCK // _NC):
            chunk = c * (_NCK // _NC) + cc
            base_row = chunk * _CH
            r0 = s * _SHARE
            _row_copy_loop(z_hbm, acc, _SHARE, 0, r0, src_advances=False)
            plsc.subcore_barrier()
            for li in range(_NW // _NS):
                t_src = s * (_NW // _NS) + li
                cv = cntv[t_src]
                npad = jnp.max(jnp.where(iota == chunk, cv, 0))
                nblk = npad // _K
                lrow = (t_src * _NCK + chunk) * _CAPB

                def win_body(w, _):
                    wb0 = w * _W
                    jb = jnp.minimum(_W, nblk - wb0)
                    pltpu.sync_copy(sl_hbm.at[pl.ds(lrow + wb0, _W)], sidx)
                    pltpu.sync_copy(dl_hbm.at[pl.ds(lrow + wb0, _W)], didx)
                    def blk_body(j, _):
                        pltpu.async_copy(h_hbm.at[sidx.at[j]], rows0,
                                         sem0).wait()
                        pltpu.sync_copy(rows0, acc.at[didx.at[j]], add=True)
                        return 0

                    lax.fori_loop(0, jb, blk_body, 0)
                    return 0

                lax.fori_loop(0, (nblk + _W - 1) // _W, win_body, 0)
            plsc.subcore_barrier()
            _row_copy_loop(acc, agg_hbm, _SHARE, r0, base_row + r0)
            plsc.subcore_barrier()

    return agg


_NFULL = _N // _K
_NTAIL = _N - _NFULL * _K


@functools.partial(
    pl.kernel,
    out_type=jax.ShapeDtypeStruct((_NC * _G, _DH), jnp.float32),
    mesh=_mesh,
    compiler_params=pltpu.CompilerParams(needs_layout_passes=False),
    scratch_types=[
        pltpu.VMEM((_K,), jnp.int32),
        pltpu.VMEM((_NTAIL,), jnp.int32),
        pltpu.VMEM((_K, _DH), jnp.float32),
        pltpu.VMEM((_NTAIL, _DH), jnp.float32),
        pltpu.VMEM_SHARED((_G, _DH), jnp.float32),
        pltpu.SemaphoreType.DMA,
    ],
)
def _pool(h_hbm, b_hbm, z_hbm, p_hbm, bidx, bidxt, rows, rowst, acc, sem):
    c = lax.axis_index("c")
    s = lax.axis_index("s")
    wid = s * _NC + c
    gshare = _G // _NS
    pltpu.sync_copy(z_hbm.at[pl.ds(0, gshare)], acc.at[pl.ds(s * gshare, gshare)])
    plsc.subcore_barrier()
    nblk = (_NFULL - wid + _NW - 1) // _NW

    def body(i, _):
        r0 = (i * _NW + wid) * _K
        pltpu.sync_copy(b_hbm.at[pl.ds(r0, _K)], bidx)
        pltpu.sync_copy(h_hbm.at[pl.ds(r0, _K)], rows)
        pltpu.sync_copy(rows, acc.at[bidx], add=True)
        return 0

    lax.fori_loop(0, nblk, body, 0)

    @pl.when(wid == 0)
    def _():
        pltpu.sync_copy(b_hbm.at[pl.ds(_NFULL * _K, _NTAIL)], bidxt)
        pltpu.sync_copy(h_hbm.at[pl.ds(_NFULL * _K, _NTAIL)], rowst)
        pltpu.sync_copy(rowst, acc.at[bidxt], add=True)

    plsc.subcore_barrier()
    pltpu.sync_copy(acc.at[pl.ds(s * gshare, gshare)],
                    p_hbm.at[pl.ds(c * _G + s * gshare, gshare)])


_RB = 2000
_NBLK = _N // _RB


@functools.lru_cache(None)
def _make_layer_a(din):
    def body(h_ref, a_ref, w_ref, b_ref, y_ref, s_ref, q_ref):
        t = h_ref[...] + a_ref[...]
        y = jnp.dot(t, w_ref[...], preferred_element_type=jnp.float32) + b_ref[...]
        y_ref[...] = y

        @pl.when(pl.program_id(0) == 0)
        def _():
            s_ref[...] = jnp.zeros_like(s_ref)
            q_ref[...] = jnp.zeros_like(q_ref)

        s_ref[...] += jnp.sum(y, axis=0, keepdims=True)
        q_ref[...] += jnp.sum(y * y, axis=0, keepdims=True)

    return pl.pallas_call(
        body,
        grid=(_NBLK,),
        in_specs=[
            pl.BlockSpec((_RB, din), lambda i: (i, 0)),
            pl.BlockSpec((_RB, din), lambda i: (i, 0)),
            pl.BlockSpec((din, _DH), lambda i: (0, 0)),
            pl.BlockSpec((1, _DH), lambda i: (0, 0)),
        ],
        out_specs=[
            pl.BlockSpec((_RB, _DH), lambda i: (i, 0)),
            pl.BlockSpec((1, _DH), lambda i: (0, 0)),
            pl.BlockSpec((1, _DH), lambda i: (0, 0)),
        ],
        out_shape=[
            jax.ShapeDtypeStruct((_N, _DH), jnp.float32),
            jax.ShapeDtypeStruct((1, _DH), jnp.float32),
            jax.ShapeDtypeStruct((1, _DH), jnp.float32),
        ],
    )


def _layer_b_body(y_ref, s_ref, q_ref, g_ref, be_ref, w_ref, bb_ref, o_ref):
    m = s_ref[...] / _N
    v = q_ref[...] / _N - m * m
    yn = (y_ref[...] - m) / jnp.sqrt(v + 1e-5) * g_ref[...] + be_ref[...]
    r = jnp.maximum(yn, 0.0)
    h2 = jnp.dot(r, w_ref[...], preferred_element_type=jnp.float32) + bb_ref[...]
    o_ref[...] = jnp.maximum(h2, 0.0)


_layer_b = pl.pallas_call(
    _layer_b_body,
    grid=(_NBLK,),
    in_specs=[
        pl.BlockSpec((_RB, _DH), lambda i: (i, 0)),
        pl.BlockSpec((1, _DH), lambda i: (0, 0)),
        pl.BlockSpec((1, _DH), lambda i: (0, 0)),
        pl.BlockSpec((1, _DH), lambda i: (0, 0)),
        pl.BlockSpec((1, _DH), lambda i: (0, 0)),
        pl.BlockSpec((_DH, _DH), lambda i: (0, 0)),
        pl.BlockSpec((1, _DH), lambda i: (0, 0)),
    ],
    out_specs=pl.BlockSpec((_RB, _DH), lambda i: (i, 0)),
    out_shape=jax.ShapeDtypeStruct((_N, _DH), jnp.float32),
)


def _head_body(p_ref, w1_ref, b1_ref, w2_ref, b2_ref, o_ref):
    hg = p_ref[:_G, :] + p_ref[_G:, :]
    z = jnp.maximum(
        jnp.dot(hg, w1_ref[...], preferred_element_type=jnp.float32) + b1_ref[...],
        0.0)
    o_ref[...] = jnp.dot(z, w2_ref[...], preferred_element_type=jnp.float32) + b2_ref[...]


_head = pl.pallas_call(
    _head_body,
    out_shape=jax.ShapeDtypeStruct((_G, _DH), jnp.float32),
)


def kernel(x, edge_index, batch,
           w1a, b1a, g1, be1, w1b, b1b,
           w2a, b2a, g2, be2, w2b, b2b,
           w3a, b3a, g3, be3, w3b, b3b,
           wl1, bl1, wl2, bl2):
    src = edge_index[0]
    dst = edge_index[1]
    x_pad = jnp.pad(x, ((0, 0), (0, _DH - 11)))
    w1a_p = jnp.pad(w1a, ((0, _DH - 11), (0, 0)))
    z128 = jnp.zeros((_K, _DH), jnp.float32)

    sl, dl, cnts = _bucket(src, dst)

    sl2 = sl.reshape(-1, _K)
    dl2 = dl.reshape(-1, _K)

    def gin_layer(h, din, wa, ba, g, be, wb, bb, z):
        aggv = _make_agg(din)(h, sl2, dl2, cnts, z)
        y, sv, qv = _make_layer_a(din)(h, aggv, wa, ba.reshape(1, -1))
        return _layer_b(y, sv, qv, g.reshape(1, -1), be.reshape(1, -1),
                        wb, bb.reshape(1, -1))

    h1 = gin_layer(x_pad, _DH, w1a_p, b1a, g1, be1, w1b, b1b, z128)
    h2 = gin_layer(h1, _DH, w2a, b2a, g2, be2, w2b, b2b, z128)
    h3 = gin_layer(h2, _DH, w3a, b3a, g3, be3, w3b, b3b, z128)

    p = _pool(h3, batch, z128)
    wl2_p = jnp.pad(wl2, ((0, 0), (0, _DH - 1)))
    bl2_p = jnp.pad(bl2, ((0, _DH - 1))).reshape(1, -1)
    o = _head(p, wl1, bl1.reshape(1, -1), wl2_p, bl2_p)
    return o[:, 0:1]


# speedup vs baseline: 1.0691x; 1.0209x over previous
import functools

import jax
import jax.numpy as jnp
from jax import lax
from jax.experimental import pallas as pl
from jax.experimental.pallas import tpu as pltpu
from jax.experimental.pallas import tpu_sc as plsc

_N = 50000
_E = 800000
_DH = 128
_G = 512
_NC = 2
_NS = 16
_NW = _NC * _NS
_EPT = _E // _NW
_CH = 12544
_NCK = 4
_NPAD = _NCK * _CH
_CAP = 26624
_K = 128
_SHARE = _CH // _NS

_mesh = plsc.VectorSubcoreMesh(core_axis_name="c", subcore_axis_name="s")


def _row_copy_loop(src_ref, dst_ref, n, src_base, dst_base, src_advances=True):
    off = 0
    while off < n:
        blk = min(128, n - off)
        soff = src_base + off if src_advances else src_base
        pltpu.sync_copy(src_ref.at[pl.ds(soff, blk)],
                        dst_ref.at[pl.ds(dst_base + off, blk)])
        off += blk


@functools.partial(
    pl.kernel,
    out_type=(
        jax.ShapeDtypeStruct((_NW * _NCK * _CAP,), jnp.int32),
        jax.ShapeDtypeStruct((_NW * _NCK * _CAP,), jnp.int32),
        jax.ShapeDtypeStruct((_NW, 16), jnp.int32),
    ),
    mesh=_mesh,
    compiler_params=pltpu.CompilerParams(needs_layout_passes=False),
    scratch_types=[
        pltpu.VMEM((_EPT + 16,), jnp.int32),
        pltpu.VMEM((_EPT + 16,), jnp.int32),
        pltpu.VMEM((_CAP + 128,), jnp.int32),
        pltpu.VMEM((_CAP + 128,), jnp.int32),
        pltpu.VMEM((16,), jnp.int32),
    ],
)
def _bucket(src_hbm, dst_hbm, sl_hbm, dl_hbm, cnt_hbm,
            srcv, dstv, sbuf, dbuf, cntv):
    c = lax.axis_index("c")
    s = lax.axis_index("s")
    wid = s * _NC + c
    base_e = wid * _EPT
    pltpu.sync_copy(src_hbm.at[pl.ds(base_e, _EPT)], srcv.at[pl.ds(0, _EPT)])
    pltpu.sync_copy(dst_hbm.at[pl.ds(base_e, _EPT)], dstv.at[pl.ds(0, _EPT)])
    iota = lax.iota(jnp.int32, 16)
    cnts = jnp.zeros((16,), jnp.int32)
    nfull = _EPT // 16
    ntail = _EPT - nfull * 16

    for b in range(_NCK):
        def scan_step(cur, d, sv, extra_mask):
            m = (d >= b * _CH) & (d < (b + 1) * _CH) & extra_mask
            plsc.store_compressed(sbuf.at[pl.ds(cur, 16)], sv, mask=m)
            plsc.store_compressed(dbuf.at[pl.ds(cur, 16)], d - b * _CH, mask=m)
            return cur + jnp.max(plsc.all_reduce_population_count(m))

        def scan_body(i, cur):
            d = dstv[pl.ds(i * 16, 16)]
            sv = srcv[pl.ds(i * 16, 16)]
            return scan_step(cur, d, sv, jnp.full((16,), True))

        cnt = lax.fori_loop(0, nfull, scan_body, jnp.int32(0))
        d = dstv[pl.ds(nfull * 16, 16)]
        sv = srcv[pl.ds(nfull * 16, 16)]
        cnt = scan_step(cnt, d, sv, iota < ntail)
        npad = ((cnt + _K - 1) // _K) * _K
        dummy_s = jnp.zeros((16,), jnp.int32)
        dummy_d = jnp.full((16,), _CH, jnp.int32)

        def pad_body(j, _):
            offp = cnt + j * 16
            k = jnp.clip(npad - offp, 0, 16)
            m = iota < k
            plsc.store_compressed(sbuf.at[pl.ds(offp, 16)], dummy_s, mask=m)
            plsc.store_compressed(dbuf.at[pl.ds(offp, 16)], dummy_d, mask=m)
            return 0

        lax.fori_loop(0, _K // 16, pad_body, 0)
        lbase = (wid * _NCK + b) * _CAP

        nbig = npad // 1024

        def dma_big(i, _):
            pltpu.sync_copy(sbuf.at[pl.ds(i * 1024, 1024)],
                            sl_hbm.at[pl.ds(lbase + i * 1024, 1024)])
            pltpu.sync_copy(dbuf.at[pl.ds(i * 1024, 1024)],
                            dl_hbm.at[pl.ds(lbase + i * 1024, 1024)])
            return 0

        lax.fori_loop(0, nbig, dma_big, 0)

        def dma_tail(i, _):
            off2 = nbig * 1024 + i * _K
            pltpu.sync_copy(sbuf.at[pl.ds(off2, _K)],
                            sl_hbm.at[pl.ds(lbase + off2, _K)])
            pltpu.sync_copy(dbuf.at[pl.ds(off2, _K)],
                            dl_hbm.at[pl.ds(lbase + off2, _K)])
            return 0

        lax.fori_loop(0, (npad - nbig * 1024) // _K, dma_tail, 0)
        cnts = jnp.where(iota == b, npad, cnts)

    cntv[...] = cnts
    pltpu.sync_copy(cntv, cnt_hbm.at[wid])


_CAPB = _CAP // _K
_W = 16


@functools.lru_cache(None)
def _make_agg(d):
    @functools.partial(
        pl.kernel,
        out_type=jax.ShapeDtypeStruct((_NPAD, d), jnp.float32),
        mesh=_mesh,
        compiler_params=pltpu.CompilerParams(needs_layout_passes=False),
        scratch_types=[
            pltpu.VMEM((_NW, 16), jnp.int32),
            pltpu.VMEM((_W, _K), jnp.int32),
            pltpu.VMEM((_W, _K), jnp.int32),
            pltpu.VMEM((_K, d), jnp.float32),
            pltpu.VMEM_SHARED((_CH + 1, d), jnp.float32),
            pltpu.SemaphoreType.DMA,
        ],
    )
    def agg(h_hbm, sl_hbm, dl_hbm, cnt_hbm, z_hbm, agg_hbm,
            cntv, sidx, didx, rows0, acc, sem0):
        c = lax.axis_index("c")
        s = lax.axis_index("s")
        iota = lax.iota(jnp.int32, 16)
        pltpu.sync_copy(cnt_hbm, cntv)
        for cc in range(_NCK // _NC):
            chunk = c * (_NCK // _NC) + cc
            base_row = chunk * _CH
            r0 = s * _SHARE
            pltpu.sync_copy(z_hbm, rows0)
            _row_copy_loop(rows0, acc, _SHARE, 0, r0, src_advances=False)
            plsc.subcore_barrier()
            for li in range(_NW // _NS):
                t_src = s * (_NW // _NS) + li
                cv = cntv[t_src]
                npad = jnp.max(jnp.where(iota == chunk, cv, 0))
                nblk = npad // _K
                lrow = (t_src * _NCK + chunk) * _CAPB

                def win_body(w, _):
                    wb0 = w * _W
                    jb = jnp.minimum(_W, nblk - wb0)
                    pltpu.sync_copy(sl_hbm.at[pl.ds(lrow + wb0, _W)], sidx)
                    pltpu.sync_copy(dl_hbm.at[pl.ds(lrow + wb0, _W)], didx)
                    def blk_body(j, _):
                        pltpu.async_copy(h_hbm.at[sidx.at[j]], rows0,
                                         sem0).wait()
                        pltpu.sync_copy(rows0, acc.at[didx.at[j]], add=True)
                        return 0

                    lax.fori_loop(0, jb, blk_body, 0)
                    return 0

                lax.fori_loop(0, (nblk + _W - 1) // _W, win_body, 0)
            plsc.subcore_barrier()
            _row_copy_loop(acc, agg_hbm, _SHARE, r0, base_row + r0)
            plsc.subcore_barrier()

    return agg


_NFULL = _N // _K
_NTAIL = _N - _NFULL * _K


@functools.partial(
    pl.kernel,
    out_type=jax.ShapeDtypeStruct((_NC * _G, _DH), jnp.float32),
    mesh=_mesh,
    compiler_params=pltpu.CompilerParams(needs_layout_passes=False),
    scratch_types=[
        pltpu.VMEM((_K,), jnp.int32),
        pltpu.VMEM((_NTAIL,), jnp.int32),
        pltpu.VMEM((_K, _DH), jnp.float32),
        pltpu.VMEM((_NTAIL, _DH), jnp.float32),
        pltpu.VMEM_SHARED((_G, _DH), jnp.float32),
        pltpu.SemaphoreType.DMA,
    ],
)
def _pool(h_hbm, b_hbm, z_hbm, p_hbm, bidx, bidxt, rows, rowst, acc, sem):
    c = lax.axis_index("c")
    s = lax.axis_index("s")
    wid = s * _NC + c
    gshare = _G // _NS
    pltpu.sync_copy(z_hbm.at[pl.ds(0, gshare)], acc.at[pl.ds(s * gshare, gshare)])
    plsc.subcore_barrier()
    nblk = (_NFULL - wid + _NW - 1) // _NW

    def body(i, _):
        r0 = (i * _NW + wid) * _K
        pltpu.sync_copy(b_hbm.at[pl.ds(r0, _K)], bidx)
        pltpu.sync_copy(h_hbm.at[pl.ds(r0, _K)], rows)
        pltpu.sync_copy(rows, acc.at[bidx], add=True)
        return 0

    lax.fori_loop(0, nblk, body, 0)

    @pl.when(wid == 0)
    def _():
        pltpu.sync_copy(b_hbm.at[pl.ds(_NFULL * _K, _NTAIL)], bidxt)
        pltpu.sync_copy(h_hbm.at[pl.ds(_NFULL * _K, _NTAIL)], rowst)
        pltpu.sync_copy(rowst, acc.at[bidxt], add=True)

    plsc.subcore_barrier()
    pltpu.sync_copy(acc.at[pl.ds(s * gshare, gshare)],
                    p_hbm.at[pl.ds(c * _G + s * gshare, gshare)])


_RB = 2000
_NBLK = _N // _RB


@functools.lru_cache(None)
def _make_layer_a(din):
    def body(h_ref, a_ref, w_ref, b_ref, y_ref, s_ref, q_ref):
        t = h_ref[...] + a_ref[...]
        y = jnp.dot(t, w_ref[...], preferred_element_type=jnp.float32) + b_ref[...]
        y_ref[...] = y

        @pl.when(pl.program_id(0) == 0)
        def _():
            s_ref[...] = jnp.zeros_like(s_ref)
            q_ref[...] = jnp.zeros_like(q_ref)

        s_ref[...] += jnp.sum(y, axis=0, keepdims=True)
        q_ref[...] += jnp.sum(y * y, axis=0, keepdims=True)

    return pl.pallas_call(
        body,
        grid=(_NBLK,),
        in_specs=[
            pl.BlockSpec((_RB, din), lambda i: (i, 0)),
            pl.BlockSpec((_RB, din), lambda i: (i, 0)),
            pl.BlockSpec((din, _DH), lambda i: (0, 0)),
            pl.BlockSpec((1, _DH), lambda i: (0, 0)),
        ],
        out_specs=[
            pl.BlockSpec((_RB, _DH), lambda i: (i, 0)),
            pl.BlockSpec((1, _DH), lambda i: (0, 0)),
            pl.BlockSpec((1, _DH), lambda i: (0, 0)),
        ],
        out_shape=[
            jax.ShapeDtypeStruct((_N, _DH), jnp.float32),
            jax.ShapeDtypeStruct((1, _DH), jnp.float32),
            jax.ShapeDtypeStruct((1, _DH), jnp.float32),
        ],
    )


def _layer_b_body(y_ref, s_ref, q_ref, g_ref, be_ref, w_ref, bb_ref, o_ref):
    m = s_ref[...] / _N
    v = q_ref[...] / _N - m * m
    yn = (y_ref[...] - m) / jnp.sqrt(v + 1e-5) * g_ref[...] + be_ref[...]
    r = jnp.maximum(yn, 0.0)
    h2 = jnp.dot(r, w_ref[...], preferred_element_type=jnp.float32) + bb_ref[...]
    o_ref[...] = jnp.maximum(h2, 0.0)


_layer_b = pl.pallas_call(
    _layer_b_body,
    grid=(_NBLK,),
    in_specs=[
        pl.BlockSpec((_RB, _DH), lambda i: (i, 0)),
        pl.BlockSpec((1, _DH), lambda i: (0, 0)),
        pl.BlockSpec((1, _DH), lambda i: (0, 0)),
        pl.BlockSpec((1, _DH), lambda i: (0, 0)),
        pl.BlockSpec((1, _DH), lambda i: (0, 0)),
        pl.BlockSpec((_DH, _DH), lambda i: (0, 0)),
        pl.BlockSpec((1, _DH), lambda i: (0, 0)),
    ],
    out_specs=pl.BlockSpec((_RB, _DH), lambda i: (i, 0)),
    out_shape=jax.ShapeDtypeStruct((_N, _DH), jnp.float32),
)


def _head_body(p_ref, w1_ref, b1_ref, w2_ref, b2_ref, o_ref):
    hg = p_ref[:_G, :] + p_ref[_G:, :]
    z = jnp.maximum(
        jnp.dot(hg, w1_ref[...], preferred_element_type=jnp.float32) + b1_ref[...],
        0.0)
    o_ref[...] = jnp.dot(z, w2_ref[...], preferred_element_type=jnp.float32) + b2_ref[...]


_head = pl.pallas_call(
    _head_body,
    out_shape=jax.ShapeDtypeStruct((_G, _DH), jnp.float32),
)


def kernel(x, edge_index, batch,
           w1a, b1a, g1, be1, w1b, b1b,
           w2a, b2a, g2, be2, w2b, b2b,
           w3a, b3a, g3, be3, w3b, b3b,
           wl1, bl1, wl2, bl2):
    src = edge_index[0]
    dst = edge_index[1]
    x_pad = jnp.pad(x, ((0, 0), (0, _DH - 11)))
    w1a_p = jnp.pad(w1a, ((0, _DH - 11), (0, 0)))
    z128 = jnp.zeros((_K, _DH), jnp.float32)

    sl, dl, cnts = _bucket(src, dst)

    sl2 = sl.reshape(-1, _K)
    dl2 = dl.reshape(-1, _K)

    def gin_layer(h, din, wa, ba, g, be, wb, bb, z):
        aggv = _make_agg(din)(h, sl2, dl2, cnts, z)
        y, sv, qv = _make_layer_a(din)(h, aggv, wa, ba.reshape(1, -1))
        return _layer_b(y, sv, qv, g.reshape(1, -1), be.reshape(1, -1),
                        wb, bb.reshape(1, -1))

    h1 = gin_layer(x_pad, _DH, w1a_p, b1a, g1, be1, w1b, b1b, z128)
    h2 = gin_layer(h1, _DH, w2a, b2a, g2, be2, w2b, b2b, z128)
    h3 = gin_layer(h2, _DH, w3a, b3a, g3, be3, w3b, b3b, z128)

    p = _pool(h3, batch, z128)
    wl2_p = jnp.pad(wl2, ((0, 0), (0, _DH - 1)))
    bl2_p = jnp.pad(bl2, ((0, _DH - 1))).reshape(1, -1)
    o = _head(p, wl1, bl1.reshape(1, -1), wl2_p, bl2_p)
    return o[:, 0:1]
